# baseline (device time: 622814 ns/iter reference)
import jax
import jax.numpy as jnp
from jax import lax
from jax.experimental import pallas as pl
from jax.experimental.pallas import tpu as pltpu

N_DEV = 32
B = 2
SQ = 128
D = 512
HL = 8
DH = 64
SKV = 128


def kernel(x, Wq, Wo, K_ext, V_ext):
    me_out = lax.axis_index("i")
    K_loc = lax.dynamic_slice_in_dim(K_ext, me_out * HL, HL, axis=2)
    V_loc = lax.dynamic_slice_in_dim(V_ext, me_out * HL, HL, axis=2)
    K_t = jnp.transpose(K_loc, (0, 2, 1, 3))
    V_t = jnp.transpose(V_loc, (0, 2, 1, 3))

    def body(x_ref, wq_ref, wo_ref, k_ref, v_ref, out_ref,
             comm_ref, send_sems, recv_sems):
        me = lax.axis_index("i")
        left = lax.rem(me + N_DEV - 1, N_DEV)
        right = lax.rem(me + 1, N_DEV)

        barrier_sem = pltpu.get_barrier_semaphore()
        for nbr in (left, right):
            pl.semaphore_signal(
                barrier_sem, inc=1,
                device_id=(nbr,), device_id_type=pl.DeviceIdType.MESH,
            )
        pl.semaphore_wait(barrier_sem, 2)

        def contribution(slot, first):
            for b in range(B):
                xb = comm_ref[slot, 0, b]
                q = jnp.dot(xb, wq_ref[...],
                            preferred_element_type=jnp.float32)
                outs = []
                for h in range(HL):
                    qh = q[:, h * DH:(h + 1) * DH]
                    kh = k_ref[b, h]
                    vh = v_ref[b, h]
                    s = lax.dot_general(
                        qh, kh, (((1,), (1,)), ((), ())),
                        preferred_element_type=jnp.float32) * 0.125
                    p = jnp.exp(s - jnp.max(s, axis=1, keepdims=True))
                    p = p / jnp.sum(p, axis=1, keepdims=True)
                    outs.append(jnp.dot(p, vh,
                                        preferred_element_type=jnp.float32))
                attn = jnp.concatenate(outs, axis=1)
                y = jnp.dot(attn, wo_ref[...],
                            preferred_element_type=jnp.float32)
                if first:
                    comm_ref[slot, 1, b] = y
                else:
                    comm_ref[slot, 1, b] = comm_ref[slot, 1, b] + y

        def send_and_wait(slot):
            rdma = pltpu.make_async_remote_copy(
                src_ref=comm_ref.at[slot],
                dst_ref=comm_ref.at[1 - slot],
                send_sem=send_sems.at[slot],
                recv_sem=recv_sems.at[1 - slot],
                device_id=(right,),
                device_id_type=pl.DeviceIdType.MESH,
            )
            rdma.start()
            rdma.wait()

        comm_ref[0, 0] = x_ref[...]
        contribution(0, first=True)
        send_and_wait(0)

        def hop(h, carry):
            slot = lax.rem(h, 2)
            contribution(slot, first=False)
            send_and_wait(slot)
            return carry

        lax.fori_loop(1, N_DEV, hop, 0)

        out_ref[...] = comm_ref[0, 1]

    return pl.pallas_call(
        body,
        out_shape=jax.ShapeDtypeStruct((B, SQ, D), jnp.float32),
        in_specs=[pl.BlockSpec(memory_space=pltpu.VMEM)] * 5,
        out_specs=pl.BlockSpec(memory_space=pltpu.VMEM),
        scratch_shapes=[
            pltpu.VMEM((2, 2, B, SQ, D), jnp.float32),
            pltpu.SemaphoreType.DMA((2,)),
            pltpu.SemaphoreType.DMA((2,)),
        ],
        compiler_params=pltpu.CompilerParams(collective_id=0),
    )(x, Wq, Wo, K_t, V_t)


# device time: 408685 ns/iter; 1.5239x vs baseline; 1.5239x over previous
import jax
import jax.numpy as jnp
from jax import lax
from jax.experimental import pallas as pl
from jax.experimental.pallas import tpu as pltpu

N_DEV = 32
B = 2
SQ = 128
D = 512
HL = 8
DH = 64
SKV = 128
S = 8

CX_CW, CX_CCW, CA_CW, CA_CCW = 0, 1, 2, 3


def kernel(x, Wq, Wo, K_ext, V_ext):
    me_out = lax.axis_index("i")
    K_loc = lax.dynamic_slice_in_dim(K_ext, me_out * HL, HL, axis=2)
    V_loc = lax.dynamic_slice_in_dim(V_ext, me_out * HL, HL, axis=2)
    K_t = jnp.transpose(K_loc, (0, 2, 1, 3))
    V_t = jnp.transpose(V_loc, (0, 2, 1, 3))

    def body(x_ref, wq_ref, wo_ref, k_ref, v_ref, out_ref,
             xb_cw, xb_ccw, ab_cw, ab_ccw,
             xs_s_cw, xs_r_cw, xs_s_ccw, xs_r_ccw,
             as_s_cw, as_r_cw, as_s_ccw, as_r_ccw, cred):
        me = lax.axis_index("i")
        left = lax.rem(me + N_DEV - 1, N_DEV)
        right = lax.rem(me + 1, N_DEV)

        barrier_sem = pltpu.get_barrier_semaphore()
        for nbr in (left, right):
            pl.semaphore_signal(
                barrier_sem, inc=1,
                device_id=(nbr,), device_id_type=pl.DeviceIdType.MESH,
            )
        pl.semaphore_wait(barrier_sem, 2)

        def rc(src, dst, ssem, rsem, peer):
            return pltpu.make_async_remote_copy(
                src_ref=src, dst_ref=dst, send_sem=ssem, recv_sem=rsem,
                device_id=(peer,), device_id_type=pl.DeviceIdType.MESH,
            )

        def send_x(t, buf, ssems, rsems, peer, src=None):
            src = buf.at[lax.rem(t, S)] if src is None else src
            rc(src, buf.at[lax.rem(t + 1, S)],
               ssems.at[lax.rem(t, S)], rsems.at[lax.rem(t + 1, S)],
               peer).start()

        def wait_recv(r, buf, ssems, rsems):
            sl = lax.rem(r, S)
            rc(buf.at[sl], buf.at[sl], ssems.at[0], rsems.at[sl],
               me).wait_recv()

        def wait_sent(t, buf, ssems, rsems):
            sl = lax.rem(t, S)
            rc(buf.at[sl], buf.at[sl], ssems.at[sl], rsems.at[0],
               me).wait_send()

        def sig(idx, peer):
            pl.semaphore_signal(
                cred.at[idx], inc=1,
                device_id=(peer,), device_id_type=pl.DeviceIdType.MESH,
            )

        def contrib_pair(x0, x1):
            xcat = jnp.concatenate([x0, x1], axis=0)
            q2 = jnp.dot(xcat, wq_ref[...],
                         preferred_element_type=jnp.float32)
            attns = []
            for b in range(B):
                qb = q2[b * SQ:(b + 1) * SQ]
                outs = []
                for h in range(HL):
                    qh = qb[:, h * DH:(h + 1) * DH]
                    s = lax.dot_general(
                        qh, k_ref[b, h], (((1,), (1,)), ((), ())),
                        preferred_element_type=jnp.float32) * 0.125
                    p = jnp.exp(s - jnp.max(s, axis=1, keepdims=True))
                    p = p / jnp.sum(p, axis=1, keepdims=True)
                    outs.append(jnp.dot(p, v_ref[b, h],
                                        preferred_element_type=jnp.float32))
                attns.append(jnp.concatenate(outs, axis=1))
            acat = jnp.concatenate(attns, axis=0)
            y2 = jnp.dot(acat, wo_ref[...],
                         preferred_element_type=jnp.float32)
            return y2[:SQ], y2[SQ:]

        send_x(0, xb_cw, xs_s_cw, xs_r_cw, right, src=x_ref.at[0])
        send_x(0, xb_ccw, xs_s_ccw, xs_r_ccw, left, src=x_ref.at[1])
        y0, y1 = contrib_pair(x_ref[0], x_ref[1])
        ab_cw[0] = y0
        send_x(0, ab_cw, as_s_cw, as_r_cw, right)
        ab_ccw[0] = y1
        send_x(0, ab_ccw, as_s_ccw, as_r_ccw, left)

        def hop(k, carry):
            sl = lax.rem(k, S)

            wait_sent(k - 1, xb_cw, xs_s_cw, xs_r_cw)
            wait_sent(k - 1, xb_ccw, xs_s_ccw, xs_r_ccw)

            @pl.when(jnp.logical_and(k >= 2, k <= 24))
            def _():
                sig(CX_CW, left)
                sig(CX_CCW, right)

            wait_sent(k - 1, ab_cw, as_s_cw, as_r_cw)
            wait_sent(k - 1, ab_ccw, as_s_ccw, as_r_ccw)

            @pl.when(jnp.logical_and(k >= 2, k <= 25))
            def _():
                sig(CA_CW, left)
                sig(CA_CCW, right)

            wait_recv(k, xb_cw, xs_s_cw, xs_r_cw)
            wait_recv(k, xb_ccw, xs_s_ccw, xs_r_ccw)

            @pl.when(jnp.logical_and(k <= 30, k >= S))
            def _():
                pl.semaphore_wait(cred.at[CX_CW], 1)
                pl.semaphore_wait(cred.at[CX_CCW], 1)

            @pl.when(k <= 30)
            def _():
                send_x(k, xb_cw, xs_s_cw, xs_r_cw, right)
                send_x(k, xb_ccw, xs_s_ccw, xs_r_ccw, left)

            y_cw, y_ccw = contrib_pair(xb_cw[sl], xb_ccw[sl])

            wait_recv(k, ab_cw, as_s_cw, as_r_cw)
            ab_cw[sl] = ab_cw[sl] + y_cw

            @pl.when(k >= S)
            def _():
                pl.semaphore_wait(cred.at[CA_CW], 1)

            send_x(k, ab_cw, as_s_cw, as_r_cw, right)

            wait_recv(k, ab_ccw, as_s_ccw, as_r_ccw)
            ab_ccw[sl] = ab_ccw[sl] + y_ccw

            @pl.when(k >= S)
            def _():
                pl.semaphore_wait(cred.at[CA_CCW], 1)

            send_x(k, ab_ccw, as_s_ccw, as_r_ccw, left)
            return carry

        lax.fori_loop(1, N_DEV, hop, 0)

        wait_sent(N_DEV - 1, ab_cw, as_s_cw, as_r_cw)
        wait_sent(N_DEV - 1, ab_ccw, as_s_ccw, as_r_ccw)
        wait_recv(N_DEV, ab_cw, as_s_cw, as_r_cw)
        out_ref[0] = ab_cw[0]
        wait_recv(N_DEV, ab_ccw, as_s_ccw, as_r_ccw)
        out_ref[1] = ab_ccw[0]

    return pl.pallas_call(
        body,
        out_shape=jax.ShapeDtypeStruct((B, SQ, D), jnp.float32),
        in_specs=[pl.BlockSpec(memory_space=pltpu.VMEM)] * 5,
        out_specs=pl.BlockSpec(memory_space=pltpu.VMEM),
        scratch_shapes=[
            pltpu.VMEM((S, SQ, D), jnp.float32),
            pltpu.VMEM((S, SQ, D), jnp.float32),
            pltpu.VMEM((S, SQ, D), jnp.float32),
            pltpu.VMEM((S, SQ, D), jnp.float32),
            pltpu.SemaphoreType.DMA((S,)),
            pltpu.SemaphoreType.DMA((S,)),
            pltpu.SemaphoreType.DMA((S,)),
            pltpu.SemaphoreType.DMA((S,)),
            pltpu.SemaphoreType.DMA((S,)),
            pltpu.SemaphoreType.DMA((S,)),
            pltpu.SemaphoreType.DMA((S,)),
            pltpu.SemaphoreType.DMA((S,)),
            pltpu.SemaphoreType.REGULAR((4,)),
        ],
        compiler_params=pltpu.CompilerParams(
            collective_id=0,
            vmem_limit_bytes=64 * 1024 * 1024,
        ),
    )(x, Wq, Wo, K_t, V_t)
